# baseline (speedup 1.0000x reference)
import functools

import jax
import jax.numpy as jnp
from jax import lax
from jax.experimental import pallas as pl
from jax.experimental.pallas import tpu as pltpu
from jax.experimental.pallas import tpu_sc as plsc

B, S, D, W = 4, 4096, 1024, 2048

NC, NS = 2, 16
NW = NC * NS
WK_PER_B = NW // B
RPW = (W + 1 + 7) // 8 // WK_PER_B * 8
CHUNK = 32
NBUF = 3
NCHUNK = RPW // CHUNK


def _build_sc_gather():
    mesh = plsc.VectorSubcoreMesh(core_axis_name="c", subcore_axis_name="s")

    @functools.partial(
        pl.kernel,
        mesh=mesh,
        out_type=jax.ShapeDtypeStruct((B, W + 1, D), jnp.float32),
        scratch_types=[
            pltpu.VMEM((16 + W + 16,), jnp.int32),
            pltpu.VMEM((RPW + 16,), jnp.int32),
            pltpu.VMEM((16,), jnp.int32),
            pltpu.VMEM((NBUF, CHUNK, D), jnp.float32),
            pltpu.VMEM((16, D), jnp.float32),
        ]
        + [pltpu.SemaphoreType.DMA] * (2 * NBUF),
    )
    def sc_gather(
        hid_hbm, widx_hbm, out_hbm, widx_v, idx_v, didx_v, rows_v, rows16_v, *sems
    ):
        gsem, osem = sems[:NBUF], sems[NBUF:]
        wid = lax.axis_index("s") * NC + lax.axis_index("c")
        b = wid // WK_PER_B
        wk = wid % WK_PER_B
        pltpu.sync_copy(widx_hbm.at[pl.ds(b * W, W)], widx_v.at[pl.ds(16, W)])
        lanes = lax.iota(jnp.int32, 16)
        widx_v[pl.ds(0, 16)] = jnp.where(lanes == 15, -1, 0)
        widx_v[pl.ds(16 + W, 16)] = jnp.zeros((16,), jnp.int32)

        base_j = wk * RPW
        hid0 = b * S
        for i in range(RPW // 16 + 1):
            v = widx_v[pl.ds(15 + base_j + i * 16, 16)]
            idx_v[pl.ds(i * 16, 16)] = v + (hid0 + 1)

        def start_gather(c, rows=CHUNK):
            s = c % NBUF
            return pltpu.async_copy(
                hid_hbm.at[idx_v.at[pl.ds(c * CHUNK, rows)]],
                rows_v.at[s, pl.ds(0, rows)],
                gsem[s],
            )

        gathers = [start_gather(c) for c in range(NBUF)]
        writes = [None] * NBUF
        for c in range(NCHUNK):
            s = c % NBUF
            gathers[s].wait()
            writes[s] = pltpu.async_copy(
                rows_v.at[s],
                out_hbm.at[b, pl.ds(base_j + c * CHUNK, CHUNK)],
                osem[s],
            )
            if c + NBUF < NCHUNK:
                writes[s].wait()
                gathers[s] = start_gather(c + NBUF)
        for c in range(max(NCHUNK - NBUF, 0), NCHUNK):
            writes[c % NBUF].wait()

        @pl.when(wk == WK_PER_B - 1)
        def _():
            s_tail = widx_v[pl.ds(16 + W - 16, 16)][15]
            didx_v[pl.ds(0, 16)] = jnp.full((16,), W, jnp.int32)
            idx_v[pl.ds(RPW, 16)] = jnp.full((16,), hid0 + 1 + s_tail, jnp.int32)
            sl = pltpu.async_copy(
                hid_hbm.at[idx_v.at[pl.ds(RPW, 16)]], rows16_v, gsem[0]
            )
            sl.wait()
            pltpu.async_copy(rows16_v, out_hbm.at[b].at[didx_v], osem[0]).wait()

    return sc_gather


_sc_gather = _build_sc_gather()


def kernel(hidden, word_index, word_attention_mask):
    hid_flat = hidden.reshape(B * S, D)
    widx_flat = word_index.astype(jnp.int32).reshape(B * W)
    out = _sc_gather(hid_flat, widx_flat)
    return out, word_attention_mask


# speedup vs baseline: 1.8604x; 1.8604x over previous
import functools

import jax
import jax.numpy as jnp
from jax import lax
from jax.experimental import pallas as pl
from jax.experimental.pallas import tpu as pltpu
from jax.experimental.pallas import tpu_sc as plsc

B, S, D, W = 4, 4096, 1024, 2048

NC, NS = 2, 16
NW = NC * NS
WPW = W // NW
R = 8
WSEG = 16 + W + 16


def _build_sc_gather():
    mesh = plsc.VectorSubcoreMesh(core_axis_name="c", subcore_axis_name="s")

    @functools.partial(
        pl.kernel,
        mesh=mesh,
        out_type=jax.ShapeDtypeStruct((W + 1, B, D), jnp.float32),
        scratch_types=[
            pltpu.VMEM((B * WSEG,), jnp.int32),
            pltpu.VMEM((8 * (WPW + 2),), jnp.int32),
        ]
        + [pltpu.SemaphoreType.DMA] * (2 * R)
        + [pltpu.VMEM((B, D), jnp.float32)] * R,
    )
    def sc_gather(hid_hbm, widx_hbm, out_hbm, widx_v, idx_v, *rest):
        gsem, osem, bufs = rest[:R], rest[R : 2 * R], rest[2 * R :]
        wid = lax.axis_index("s") * NC + lax.axis_index("c")
        w0 = wid * WPW
        lanes = lax.iota(jnp.int32, 16)
        for b in range(B):
            pltpu.sync_copy(
                widx_hbm.at[pl.ds(b * W, W)], widx_v.at[pl.ds(b * WSEG + 16, W)]
            )
            widx_v[pl.ds(b * WSEG, 16)] = jnp.where(lanes == 15, -1, 0)
            widx_v[pl.ds(b * WSEG + 16 + W, 16)] = jnp.zeros((16,), jnp.int32)

        sel = lanes % 8
        half = lanes < 8
        for m in range(WPW // 2 + 1):
            acc = jnp.zeros((16,), jnp.int32)
            for b in range(B):
                va = widx_v[pl.ds(b * WSEG + 15 + w0 + m * 2, 16)]
                vv = jnp.where(half, va[0], va[1]) + (b * S + 1)
                acc = jnp.where(sel == b, vv, acc)
            idx_v[pl.ds(m * 16, 16)] = acc

        def start_gather(j):
            r = j % R
            return pltpu.async_copy(
                hid_hbm.at[idx_v.at[pl.ds(j * 8, B)]], bufs[r], gsem[r]
            )

        gathers = [start_gather(j) for j in range(R)]
        writes = [None] * R
        for j in range(WPW):
            r = j % R
            gathers[r].wait()
            writes[r] = pltpu.async_copy(bufs[r], out_hbm.at[w0 + j], osem[r])
            if j + R < WPW:
                writes[r].wait()
                gathers[r] = start_gather(j + R)
        for j in range(max(WPW - R, 0), WPW):
            writes[j % R].wait()

        @pl.when(wid == NW - 1)
        def _():
            sl = pltpu.async_copy(
                hid_hbm.at[idx_v.at[pl.ds(8 * WPW, B)]], bufs[0], gsem[0]
            )
            sl.wait()
            pltpu.sync_copy(bufs[0], out_hbm.at[W])

    return sc_gather


_sc_gather = _build_sc_gather()


def kernel(hidden, word_index, word_attention_mask):
    hid_flat = hidden.reshape(B * S, D)
    widx_flat = word_index.astype(jnp.int32).reshape(B * W)
    out_wmajor = _sc_gather(hid_flat, widx_flat)
    return out_wmajor.transpose(1, 0, 2), word_attention_mask


# speedup vs baseline: 1.9383x; 1.0418x over previous
import functools

import jax
import jax.numpy as jnp
from jax import lax
from jax.experimental import pallas as pl
from jax.experimental.pallas import tpu as pltpu
from jax.experimental.pallas import tpu_sc as plsc

B, S, D, W = 4, 4096, 1024, 2048

NC, NS = 2, 16
NW = NC * NS
WPW = W // NW
R = 8
WSEG = 16 + W + 16


def _build_sc_gather():
    mesh = plsc.VectorSubcoreMesh(core_axis_name="c", subcore_axis_name="s")

    @functools.partial(
        pl.kernel,
        mesh=mesh,
        out_type=jax.ShapeDtypeStruct((W + 1, B, D), jnp.float32),
        scratch_types=[
            pltpu.VMEM((B * WSEG,), jnp.int32),
            pltpu.VMEM((8 * (WPW + 2),), jnp.int32),
        ]
        + [pltpu.SemaphoreType.DMA] * (2 * R)
        + [pltpu.VMEM((B, D), jnp.float32)] * R,
    )
    def sc_gather(hid_hbm, widx_hbm, out_hbm, widx_v, idx_v, *rest):
        gsem, osem, bufs = rest[:R], rest[R : 2 * R], rest[2 * R :]
        wid = lax.axis_index("s") * NC + lax.axis_index("c")
        w0 = wid * WPW
        lanes = lax.iota(jnp.int32, 16)
        stage = [
            pltpu.async_copy(
                widx_hbm.at[pl.ds(b * W, W)],
                widx_v.at[pl.ds(b * WSEG + 16, W)],
                gsem[b],
            )
            for b in range(B)
        ]
        for b in range(B):
            widx_v[pl.ds(b * WSEG, 16)] = jnp.where(lanes == 15, -1, 0)
            widx_v[pl.ds(b * WSEG + 16 + W, 16)] = jnp.zeros((16,), jnp.int32)
        for c in stage:
            c.wait()

        sel = lanes % 8
        half = lanes < 8

        def compute_group(m):
            acc = jnp.zeros((16,), jnp.int32)
            for b in range(B):
                va = widx_v[pl.ds(b * WSEG + 15 + w0 + m * 2, 16)]
                vv = jnp.where(half, va[0], va[1]) + (b * S + 1)
                acc = jnp.where(sel == b, vv, acc)
            idx_v[pl.ds(m * 16, 16)] = acc

        def start_gather(j):
            r = j % R
            return pltpu.async_copy(
                hid_hbm.at[idx_v.at[pl.ds(j * 8, B)]], bufs[r], gsem[r]
            )

        for m in range(R // 2):
            compute_group(m)
        gathers = [start_gather(j) for j in range(R)]
        for m in range(R // 2, WPW // 2 + 1):
            compute_group(m)
        writes = [None] * R
        for j in range(WPW):
            r = j % R
            gathers[r].wait()
            writes[r] = pltpu.async_copy(bufs[r], out_hbm.at[w0 + j], osem[r])
            if j + R < WPW:
                writes[r].wait()
                gathers[r] = start_gather(j + R)
        for j in range(max(WPW - R, 0), WPW):
            writes[j % R].wait()

        @pl.when(wid == NW - 1)
        def _():
            sl = pltpu.async_copy(
                hid_hbm.at[idx_v.at[pl.ds(8 * WPW, B)]], bufs[0], gsem[0]
            )
            sl.wait()
            pltpu.sync_copy(bufs[0], out_hbm.at[W])

    return sc_gather


_sc_gather = _build_sc_gather()


def kernel(hidden, word_index, word_attention_mask):
    hid_flat = hidden.reshape(B * S, D)
    widx_flat = word_index.astype(jnp.int32).reshape(B * W)
    out_wmajor = _sc_gather(hid_flat, widx_flat)
    return out_wmajor.transpose(1, 0, 2), word_attention_mask
